# baseline (speedup 1.0000x reference)
import functools

import jax
import jax.numpy as jnp
from jax import lax
from jax.experimental import pallas as pl
from jax.experimental.pallas import tpu as pltpu
from jax.experimental.pallas import tpu_sc as plsc

N = 10000
E = 320000
HD = 128
H = 8
OUT = 16
NEG = 0.2

RB = 1000
NRB = N // RB

NC = 2
NS = 16
NW = NC * NS
EPW = E // NW
CH = 40
NCHUNK = EPW // CH
DEPTH = 5
NPAD = 10240
RPT = NPAD // NS
CPR = 128
AW = HD + 16
FW = 160
HDT = jnp.bfloat16


def _proj_and_shift(feat, alv, arv, S16, elp_ref, rp_ref, c_ref, mx_ref, i):
    el = jnp.dot(feat * alv, S16, preferred_element_type=jnp.float32)
    er = jnp.dot(feat * arv, S16, preferred_element_type=jnp.float32)
    elp_ref[...] = el
    rp_ref[...] = er
    bmax0 = jnp.max(el, axis=0, keepdims=True)
    bmax1 = jnp.max(er, axis=0, keepdims=True)

    @pl.when(i == 0)
    def _():
        mx_ref[0:1] = bmax0
        mx_ref[1:2] = bmax1

    @pl.when(i > 0)
    def _():
        mx_ref[0:1] = jnp.maximum(mx_ref[0:1], bmax0)
        mx_ref[1:2] = jnp.maximum(mx_ref[1:2], bmax1)

    @pl.when(i == NRB - 1)
    def _():
        s = mx_ref[0:1] + mx_ref[1:2]
        c_ref[...] = jnp.where(s > 0, s, s * NEG)


def _pre_body(x_ref, W_ref, alv_ref, arv_ref, S16_ref,
              feat_ref, elp_ref, rp_ref, c_ref, mx_ref):
    i = pl.program_id(0)
    feat = jnp.dot(x_ref[...], W_ref[...], preferred_element_type=jnp.float32)
    feat_ref[...] = feat
    _proj_and_shift(feat, alv_ref[...], arv_ref[...], S16_ref[...],
                    elp_ref, rp_ref, c_ref, mx_ref, i)


def _combine(p0, p1, hprev, bvec, ST):
    r = p0 + p1
    acc = r[:, 0:HD]
    den = r[:, HD:AW]
    inv = 1.0 / (den + 1e-30)
    inv128 = jnp.dot(inv, ST, preferred_element_type=jnp.float32)
    return acc * inv128 + hprev + bvec


def _mid_body(p0_ref, p1_ref, hprev_ref, bvec_ref, W_ref, alv_ref, arv_ref,
              S16_ref, ST_ref,
              feat_ref, elp_ref, rp_ref, c_ref, hn_ref, mx_ref):
    i = pl.program_id(0)
    pre = _combine(p0_ref[...], p1_ref[...], hprev_ref[...], bvec_ref[...],
                   ST_ref[...])
    hn = jnp.where(pre > 0, pre, jnp.exp(jnp.minimum(pre, 0.0)) - 1.0)
    hn_ref[...] = hn
    feat = jnp.dot(hn, W_ref[...], preferred_element_type=jnp.float32)
    feat_ref[...] = feat
    _proj_and_shift(feat, alv_ref[...], arv_ref[...], S16_ref[...],
                    elp_ref, rp_ref, c_ref, mx_ref, i)


def _post_body(p0_ref, p1_ref, h2_ref, bvec_ref, ST_ref, G_ref, out_ref):
    final = _combine(p0_ref[...], p1_ref[...], h2_ref[...], bvec_ref[...],
                     ST_ref[...])
    out_ref[...] = jnp.dot(final, G_ref[...], preferred_element_type=jnp.float32)


def _row_spec(cols):
    return pl.BlockSpec((RB, cols), lambda i: (i, 0))


def _full_spec(rows, cols):
    return pl.BlockSpec((rows, cols), lambda i: (0, 0))


_F32 = jnp.float32


def _tc_pre(x, W, alv, arv, S16):
    return pl.pallas_call(
        _pre_body,
        grid=(NRB,),
        in_specs=[_row_spec(HD), _full_spec(HD, HD), _full_spec(1, HD),
                  _full_spec(1, HD), _full_spec(HD, 16)],
        out_specs=[_row_spec(HD), _row_spec(16), _row_spec(16),
                   _full_spec(1, 16)],
        out_shape=[jax.ShapeDtypeStruct((N, HD), _F32),
                   jax.ShapeDtypeStruct((N, 16), _F32),
                   jax.ShapeDtypeStruct((N, 16), _F32),
                   jax.ShapeDtypeStruct((1, 16), _F32)],
        scratch_shapes=[pltpu.VMEM((2, 16), _F32)],
    )(x, W, alv, arv, S16)


def _tc_mid(p0, p1, hprev, bvec, W, alv, arv, S16, ST):
    return pl.pallas_call(
        _mid_body,
        grid=(NRB,),
        in_specs=[_row_spec(AW), _row_spec(AW), _row_spec(HD),
                  _full_spec(1, HD), _full_spec(HD, HD), _full_spec(1, HD),
                  _full_spec(1, HD), _full_spec(HD, 16), _full_spec(16, HD)],
        out_specs=[_row_spec(HD), _row_spec(16), _row_spec(16),
                   _full_spec(1, 16), _row_spec(HD)],
        out_shape=[jax.ShapeDtypeStruct((N, HD), _F32),
                   jax.ShapeDtypeStruct((N, 16), _F32),
                   jax.ShapeDtypeStruct((N, 16), _F32),
                   jax.ShapeDtypeStruct((1, 16), _F32),
                   jax.ShapeDtypeStruct((N, HD), _F32)],
        scratch_shapes=[pltpu.VMEM((2, 16), _F32)],
    )(p0, p1, hprev, bvec, W, alv, arv, S16, ST)


def _tc_post(p0, p1, h2, bvec, ST, G):
    return pl.pallas_call(
        _post_body,
        grid=(NRB,),
        in_specs=[_row_spec(AW), _row_spec(AW), _row_spec(HD),
                  _full_spec(1, HD), _full_spec(16, HD), _full_spec(HD, 16)],
        out_specs=_row_spec(16),
        out_shape=jax.ShapeDtypeStruct((N, 16), _F32),
    )(p0, p1, h2, bvec, ST, G)


def _sc_body(F_hbm, R_hbm, C_hbm, src_hbm, dst_hbm, P_hbm,
             idx_s, idx_d, sidx, Fc, Rc, M, cvb, A,
             semf0, semf1, semf2, semf3, semf4,
             semr0, semr1, semr2, semr3, semr4,
             semi0, semi1, semi2, semi3, semi4):
    cid = lax.axis_index("c")
    sid = lax.axis_index("s")
    wid = sid * NC + cid
    semf = (semf0, semf1, semf2, semf3, semf4)
    semr = (semr0, semr1, semr2, semr3, semr4)
    semi = (semi0, semi1, semi2, semi3, semi4)

    def _zrow(i, _):
        for j in range(AW // 16):
            M[i, pl.ds(j * 16, 16)] = jnp.zeros((16,), _F32)
        return 0

    lax.fori_loop(0, CH, _zrow, 0)
    for j in range(RPT // CH):
        pltpu.sync_copy(M, A.at[pl.ds(sid * RPT + j * CH, CH)])
    plsc.subcore_barrier()

    pltpu.sync_copy(C_hbm, cvb)
    cv = cvb[...]

    base0 = wid * EPW

    def _fetch_idx_sync(c, b):
        base = base0 + c * CH
        pltpu.sync_copy(src_hbm.at[pl.ds(base, CH)], idx_s.at[b])
        pltpu.sync_copy(dst_hbm.at[pl.ds(base, CH)], idx_d.at[b])

    def _fetch_idx_async(c, b):
        base = base0 + c * CH
        pltpu.async_copy(src_hbm.at[pl.ds(base, CH)], idx_s.at[b], semi[b])
        pltpu.async_copy(dst_hbm.at[pl.ds(base, CH)], idx_d.at[b], semi[b])

    def _wait_idx(b):
        pltpu.make_async_copy(src_hbm.at[pl.ds(0, CH)], idx_s.at[b],
                              semi[b]).wait()
        pltpu.make_async_copy(dst_hbm.at[pl.ds(0, CH)], idx_d.at[b],
                              semi[b]).wait()

    def _gather_rows(b):
        pltpu.async_copy(F_hbm.at[idx_s.at[b]], Fc.at[b], semf[b])
        pltpu.async_copy(R_hbm.at[idx_d.at[b]], Rc.at[b], semr[b])

    for b in range(DEPTH - 1):
        _fetch_idx_sync(b, b)
        _gather_rows(b)
    _fetch_idx_async(DEPTH - 1, DEPTH - 1)

    def _phase(g, _):
        for b in range(DEPTH):
            c = DEPTH * g + b
            bp = (b - 1) % DEPTH
            pltpu.make_async_copy(F_hbm.at[idx_s.at[b]], Fc.at[b],
                                  semf[b]).wait()
            pltpu.make_async_copy(R_hbm.at[idx_d.at[b]], Rc.at[b],
                                  semr[b]).wait()
            sidx[0, pl.ds(0, 16)] = idx_d[b, pl.ds(0, 16)]
            sidx[0, pl.ds(16, 16)] = idx_d[b, pl.ds(16, 16)]
            sidx[0, pl.ds(24, 16)] = idx_d[b, pl.ds(24, 16)]

            @pl.when(c + DEPTH - 1 < NCHUNK)
            def _():
                _wait_idx(bp)
                _gather_rows(bp)

            @pl.when(c + DEPTH < NCHUNK)
            def _():
                _fetch_idx_async(c + DEPTH, b)

            def _edge(i, _):
                els, _ = plsc.unpack(Fc[b, i, pl.ds(HD, 32)],
                                     format=plsc.PackFormat.INTERLEAVED,
                                     preferred_element_type=_F32)
                erd = Rc[b, i, :]
                sv = els + erd
                e = jnp.where(sv > 0, sv, sv * NEG)
                ee = jnp.exp(e - cv)
                M[i, pl.ds(HD, 16)] = ee
                for j in range(H // 2):
                    fa, fb = plsc.unpack(Fc[b, i, pl.ds(32 * j, 32)],
                                         format=plsc.PackFormat.INTERLEAVED,
                                         preferred_element_type=_F32)
                    M[i, pl.ds(32 * j, 16)] = fa * ee[2 * j]
                    M[i, pl.ds(32 * j + 16, 16)] = fb * ee[2 * j + 1]
                return 0

            lax.fori_loop(0, CH, _edge, 0)
            pltpu.sync_copy(M, A.at[sidx.at[0]], add=True)
        return 0

    lax.fori_loop(0, NCHUNK // DEPTH, _phase, 0)
    plsc.subcore_barrier()

    for j in range(RPT // CH):
        r0 = sid * RPT + j * CH
        pltpu.sync_copy(A.at[pl.ds(r0, CH)], M)
        pltpu.sync_copy(M, P_hbm.at[cid, pl.ds(r0, CH)])


def _sc_edges(F, R, C, src, dst):
    mesh = plsc.VectorSubcoreMesh(core_axis_name="c", subcore_axis_name="s")
    k = pl.kernel(
        _sc_body,
        out_type=jax.ShapeDtypeStruct((NC, NPAD, AW), _F32),
        mesh=mesh,
        compiler_params=pltpu.CompilerParams(use_tc_tiling_on_sc=False, needs_layout_passes=False),
        scratch_types=[
            pltpu.VMEM((DEPTH, CH), jnp.int32),
            pltpu.VMEM((DEPTH, CH), jnp.int32),
            pltpu.VMEM((1, CH), jnp.int32),
            pltpu.VMEM((DEPTH, CH, FW), HDT),
            pltpu.VMEM((DEPTH, CH, 16), _F32),
            pltpu.VMEM((CH, AW), _F32),
            pltpu.VMEM((16,), _F32),
            pltpu.VMEM_SHARED((NPAD, AW), _F32),
        ] + [pltpu.SemaphoreType.DMA] * 15,
    )
    return k(F, R, C, src, dst)


def _interleave_f16(feat, elp):
    f32 = jnp.concatenate([feat, elp], axis=1)
    parts = []
    for j in range(H // 2):
        a = f32[:, 32 * j:32 * j + 16]
        b = f32[:, 32 * j + 16:32 * j + 32]
        parts.append(jnp.stack([a, b], axis=2).reshape(N, 32))
    a4 = jnp.pad(f32[:, HD:HD + H], ((0, 0), (0, 16 - H)))
    b4 = jnp.zeros((N, 16), _F32)
    parts.append(jnp.stack([a4, b4], axis=2).reshape(N, 32))
    return jnp.concatenate(parts, axis=1).astype(HDT)


def kernel(inputs, edge_index, W0, al0, ar0, b0, W1, al1, ar1, b1,
           W2, al2, ar2, b2):
    x = inputs.astype(_F32)
    src = edge_index[0].astype(jnp.int32)
    dst = edge_index[1].astype(jnp.int32)

    j = jnp.arange(HD)
    S16 = (jnp.arange(16)[None, :] == (j // OUT)[:, None]).astype(_F32)
    ST = S16.T.copy()
    G = ((jnp.arange(16)[None, :] == (j % OUT)[:, None]).astype(_F32) / H)

    def layer_consts(al, ar, b):
        return (al.reshape(1, HD), ar.reshape(1, HD), b.reshape(1, HD))

    alv0, arv0, bv0 = layer_consts(al0, ar0, b0)
    alv1, arv1, bv1 = layer_consts(al1, ar1, b1)
    alv2, arv2, bv2 = layer_consts(al2, ar2, b2)

    feat, elp, rp, C = _tc_pre(x, W0, alv0, arv0, S16)
    P = _sc_edges(_interleave_f16(feat, elp), rp, C.reshape(16), src, dst)

    zeros_h = jnp.zeros((N, HD), _F32)
    feat, elp, rp, C, h1 = _tc_mid(P[0], P[1], zeros_h, bv0, W1, alv1, arv1,
                                   S16, ST)
    P = _sc_edges(_interleave_f16(feat, elp), rp, C.reshape(16), src, dst)

    feat, elp, rp, C, h2 = _tc_mid(P[0], P[1], h1, bv1, W2, alv2, arv2,
                                   S16, ST)
    P = _sc_edges(_interleave_f16(feat, elp), rp, C.reshape(16), src, dst)

    logits = _tc_post(P[0], P[1], h2, bv2, ST, G)
    return logits


# speedup vs baseline: 1.6685x; 1.6685x over previous
import functools

import jax
import jax.numpy as jnp
from jax import lax
from jax.experimental import pallas as pl
from jax.experimental.pallas import tpu as pltpu
from jax.experimental.pallas import tpu_sc as plsc

N = 10000
E = 320000
HD = 128
H = 8
OUT = 16
NEG = 0.2

RB = 1000
NRB = N // RB

NC = 2
NS = 16
NW = NC * NS
EPW = E // NW
CH = 40
NCHUNK = EPW // CH
DEPTH = 5
NPAD = 10240
RPT = NPAD // NS
CPR = 128
AW = HD + 16


def _proj_and_shift(feat, alv, arv, S16, elp_ref, rp_ref, c_ref, mx_ref, i):
    el = jnp.dot(feat * alv, S16, preferred_element_type=jnp.float32)
    er = jnp.dot(feat * arv, S16, preferred_element_type=jnp.float32)
    elp_ref[...] = el
    rp_ref[...] = er
    bmax0 = jnp.max(el, axis=0, keepdims=True)
    bmax1 = jnp.max(er, axis=0, keepdims=True)

    @pl.when(i == 0)
    def _():
        mx_ref[0:1] = bmax0
        mx_ref[1:2] = bmax1

    @pl.when(i > 0)
    def _():
        mx_ref[0:1] = jnp.maximum(mx_ref[0:1], bmax0)
        mx_ref[1:2] = jnp.maximum(mx_ref[1:2], bmax1)

    @pl.when(i == NRB - 1)
    def _():
        s = mx_ref[0:1] + mx_ref[1:2]
        c_ref[...] = jnp.where(s > 0, s, s * NEG)


def _pre_body(x_ref, W_ref, alv_ref, arv_ref, S16_ref,
              feat_ref, elp_ref, rp_ref, c_ref, mx_ref):
    i = pl.program_id(0)
    feat = jnp.dot(x_ref[...], W_ref[...], preferred_element_type=jnp.float32)
    feat_ref[...] = feat
    _proj_and_shift(feat, alv_ref[...], arv_ref[...], S16_ref[...],
                    elp_ref, rp_ref, c_ref, mx_ref, i)


def _combine(p0, p1, hprev, bvec, ST):
    r = p0 + p1
    acc = r[:, 0:HD]
    den = r[:, HD:AW]
    inv = 1.0 / (den + 1e-30)
    inv128 = jnp.dot(inv, ST, preferred_element_type=jnp.float32)
    return acc * inv128 + hprev + bvec


def _mid_body(p0_ref, p1_ref, hprev_ref, bvec_ref, W_ref, alv_ref, arv_ref,
              S16_ref, ST_ref,
              feat_ref, elp_ref, rp_ref, c_ref, hn_ref, mx_ref):
    i = pl.program_id(0)
    pre = _combine(p0_ref[...], p1_ref[...], hprev_ref[...], bvec_ref[...],
                   ST_ref[...])
    hn = jnp.where(pre > 0, pre, jnp.exp(jnp.minimum(pre, 0.0)) - 1.0)
    hn_ref[...] = hn
    feat = jnp.dot(hn, W_ref[...], preferred_element_type=jnp.float32)
    feat_ref[...] = feat
    _proj_and_shift(feat, alv_ref[...], arv_ref[...], S16_ref[...],
                    elp_ref, rp_ref, c_ref, mx_ref, i)


def _post_body(p0_ref, p1_ref, h2_ref, bvec_ref, ST_ref, G_ref, out_ref):
    final = _combine(p0_ref[...], p1_ref[...], h2_ref[...], bvec_ref[...],
                     ST_ref[...])
    out_ref[...] = jnp.dot(final, G_ref[...], preferred_element_type=jnp.float32)


def _row_spec(cols):
    return pl.BlockSpec((RB, cols), lambda i: (i, 0))


def _full_spec(rows, cols):
    return pl.BlockSpec((rows, cols), lambda i: (0, 0))


_F32 = jnp.float32


def _tc_pre(x, W, alv, arv, S16):
    return pl.pallas_call(
        _pre_body,
        grid=(NRB,),
        in_specs=[_row_spec(HD), _full_spec(HD, HD), _full_spec(1, HD),
                  _full_spec(1, HD), _full_spec(HD, 16)],
        out_specs=[_row_spec(HD), _row_spec(16), _row_spec(16),
                   _full_spec(1, 16)],
        out_shape=[jax.ShapeDtypeStruct((N, HD), _F32),
                   jax.ShapeDtypeStruct((N, 16), _F32),
                   jax.ShapeDtypeStruct((N, 16), _F32),
                   jax.ShapeDtypeStruct((1, 16), _F32)],
        scratch_shapes=[pltpu.VMEM((2, 16), _F32)],
    )(x, W, alv, arv, S16)


def _tc_mid(p0, p1, hprev, bvec, W, alv, arv, S16, ST):
    return pl.pallas_call(
        _mid_body,
        grid=(NRB,),
        in_specs=[_row_spec(AW), _row_spec(AW), _row_spec(HD),
                  _full_spec(1, HD), _full_spec(HD, HD), _full_spec(1, HD),
                  _full_spec(1, HD), _full_spec(HD, 16), _full_spec(16, HD)],
        out_specs=[_row_spec(HD), _row_spec(16), _row_spec(16),
                   _full_spec(1, 16), _row_spec(HD)],
        out_shape=[jax.ShapeDtypeStruct((N, HD), _F32),
                   jax.ShapeDtypeStruct((N, 16), _F32),
                   jax.ShapeDtypeStruct((N, 16), _F32),
                   jax.ShapeDtypeStruct((1, 16), _F32),
                   jax.ShapeDtypeStruct((N, HD), _F32)],
        scratch_shapes=[pltpu.VMEM((2, 16), _F32)],
    )(p0, p1, hprev, bvec, W, alv, arv, S16, ST)


def _tc_post(p0, p1, h2, bvec, ST, G):
    return pl.pallas_call(
        _post_body,
        grid=(NRB,),
        in_specs=[_row_spec(AW), _row_spec(AW), _row_spec(HD),
                  _full_spec(1, HD), _full_spec(16, HD), _full_spec(HD, 16)],
        out_specs=_row_spec(16),
        out_shape=jax.ShapeDtypeStruct((N, 16), _F32),
    )(p0, p1, h2, bvec, ST, G)


def _sc_body(F_hbm, R_hbm, C_hbm, src_hbm, dst_hbm, P_hbm,
             idx_s, idx_d, sidx, Fc, Rc, M, cvb, A,
             semf0, semf1, semf2, semf3, semf4,
             semr0, semr1, semr2, semr3, semr4,
             semi0, semi1, semi2, semi3, semi4):
    cid = lax.axis_index("c")
    sid = lax.axis_index("s")
    wid = sid * NC + cid
    semf = (semf0, semf1, semf2, semf3, semf4)
    semr = (semr0, semr1, semr2, semr3, semr4)
    semi = (semi0, semi1, semi2, semi3, semi4)

    base0 = wid * EPW

    def _fetch_idx_sync(c, b):
        base = base0 + c * CH
        pltpu.sync_copy(src_hbm.at[pl.ds(base, CH)], idx_s.at[b])
        pltpu.sync_copy(dst_hbm.at[pl.ds(base, CH)], idx_d.at[b])

    def _fetch_idx_async(c, b):
        base = base0 + c * CH
        pltpu.async_copy(src_hbm.at[pl.ds(base, CH)], idx_s.at[b], semi[b])
        pltpu.async_copy(dst_hbm.at[pl.ds(base, CH)], idx_d.at[b], semi[b])

    def _wait_idx(b):
        pltpu.make_async_copy(src_hbm.at[pl.ds(0, CH)], idx_s.at[b],
                              semi[b]).wait()
        pltpu.make_async_copy(dst_hbm.at[pl.ds(0, CH)], idx_d.at[b],
                              semi[b]).wait()

    def _gather_rows(b):
        pltpu.async_copy(F_hbm.at[idx_s.at[b]], Fc.at[b], semf[b])
        pltpu.async_copy(R_hbm.at[idx_d.at[b]], Rc.at[b], semr[b])

    for b in range(DEPTH - 1):
        _fetch_idx_sync(b, b)
        _gather_rows(b)
    _fetch_idx_async(DEPTH - 1, DEPTH - 1)

    def _zrow(i, _):
        for j in range(AW // 16):
            M[i, pl.ds(j * 16, 16)] = jnp.zeros((16,), _F32)
        return 0

    lax.fori_loop(0, CH, _zrow, 0)
    for j in range(RPT // CH):
        pltpu.sync_copy(M, A.at[pl.ds(sid * RPT + j * CH, CH)])
    pltpu.sync_copy(C_hbm, cvb)
    cv = cvb[...]
    plsc.subcore_barrier()

    def _phase(g, _):
        for b in range(DEPTH):
            c = DEPTH * g + b
            bp = (b - 1) % DEPTH
            pltpu.make_async_copy(F_hbm.at[idx_s.at[b]], Fc.at[b],
                                  semf[b]).wait()
            pltpu.make_async_copy(R_hbm.at[idx_d.at[b]], Rc.at[b],
                                  semr[b]).wait()
            sidx[0, pl.ds(0, 16)] = idx_d[b, pl.ds(0, 16)]
            sidx[0, pl.ds(16, 16)] = idx_d[b, pl.ds(16, 16)]
            sidx[0, pl.ds(24, 16)] = idx_d[b, pl.ds(24, 16)]

            @pl.when(c + DEPTH - 1 < NCHUNK)
            def _():
                _wait_idx(bp)
                _gather_rows(bp)

            @pl.when(c + DEPTH < NCHUNK)
            def _():
                _fetch_idx_async(c + DEPTH, b)

            def _edge(i, _):
                els = Fc[b, i, pl.ds(HD, 16)]
                erd = Rc[b, i, :]
                sv = els + erd
                e = jnp.where(sv > 0, sv, sv * NEG)
                ee = jnp.exp(e - cv)
                M[i, pl.ds(HD, 16)] = ee
                for h in range(H):
                    M[i, pl.ds(h * OUT, OUT)] = (
                        Fc[b, i, pl.ds(h * OUT, OUT)] * ee[h])
                return 0

            lax.fori_loop(0, CH, _edge, 0)
            pltpu.sync_copy(M, A.at[sidx.at[0]], add=True)
        return 0

    lax.fori_loop(0, NCHUNK // DEPTH, _phase, 0)
    plsc.subcore_barrier()

    for j in range(RPT // CH):
        b = j % DEPTH
        r0 = sid * RPT + j * CH

        @pl.when(j >= DEPTH)
        def _():
            pltpu.make_async_copy(Fc.at[b], P_hbm.at[cid, pl.ds(0, CH)],
                                  semr[b]).wait()
        pltpu.make_async_copy(A.at[pl.ds(r0, CH)], Fc.at[b], semf[b]).start()
        pltpu.make_async_copy(A.at[pl.ds(r0, CH)], Fc.at[b], semf[b]).wait()
        pltpu.async_copy(Fc.at[b], P_hbm.at[cid, pl.ds(r0, CH)], semr[b])
    for j in range(RPT // CH - DEPTH, RPT // CH):
        b = j % DEPTH
        pltpu.make_async_copy(Fc.at[b], P_hbm.at[cid, pl.ds(0, CH)],
                              semr[b]).wait()


def _sc_edges(F, R, C, src, dst):
    mesh = plsc.VectorSubcoreMesh(core_axis_name="c", subcore_axis_name="s")
    k = pl.kernel(
        _sc_body,
        out_type=jax.ShapeDtypeStruct((NC, NPAD, AW), _F32),
        mesh=mesh,
        compiler_params=pltpu.CompilerParams(use_tc_tiling_on_sc=False),
        scratch_types=[
            pltpu.VMEM((DEPTH, CH), jnp.int32),
            pltpu.VMEM((DEPTH, CH), jnp.int32),
            pltpu.VMEM((1, CH), jnp.int32),
            pltpu.VMEM((DEPTH, CH, AW), _F32),
            pltpu.VMEM((DEPTH, CH, 16), _F32),
            pltpu.VMEM((CH, AW), _F32),
            pltpu.VMEM((16,), _F32),
            pltpu.VMEM_SHARED((NPAD, AW), _F32),
        ] + [pltpu.SemaphoreType.DMA] * 15,
    )
    return k(F, R, C, src, dst)


def kernel(inputs, edge_index, W0, al0, ar0, b0, W1, al1, ar1, b1,
           W2, al2, ar2, b2):
    x = inputs.astype(_F32)
    src = edge_index[0].astype(jnp.int32)
    dst = edge_index[1].astype(jnp.int32)

    j = jnp.arange(HD)
    S16 = (jnp.arange(16)[None, :] == (j // OUT)[:, None]).astype(_F32)
    ST = S16.T.copy()
    G = ((jnp.arange(16)[None, :] == (j % OUT)[:, None]).astype(_F32) / H)

    def layer_consts(al, ar, b):
        return (al.reshape(1, HD), ar.reshape(1, HD), b.reshape(1, HD))

    alv0, arv0, bv0 = layer_consts(al0, ar0, b0)
    alv1, arv1, bv1 = layer_consts(al1, ar1, b1)
    alv2, arv2, bv2 = layer_consts(al2, ar2, b2)

    feat, elp, rp, C = _tc_pre(x, W0, alv0, arv0, S16)
    F = jnp.concatenate([feat, elp], axis=1)
    P = _sc_edges(F, rp, C.reshape(16), src, dst)

    zeros_h = jnp.zeros((N, HD), _F32)
    feat, elp, rp, C, h1 = _tc_mid(P[0], P[1], zeros_h, bv0, W1, alv1, arv1,
                                   S16, ST)
    F = jnp.concatenate([feat, elp], axis=1)
    P = _sc_edges(F, rp, C.reshape(16), src, dst)

    feat, elp, rp, C, h2 = _tc_mid(P[0], P[1], h1, bv1, W2, alv2, arv2,
                                   S16, ST)
    F = jnp.concatenate([feat, elp], axis=1)
    P = _sc_edges(F, rp, C.reshape(16), src, dst)

    logits = _tc_post(P[0], P[1], h2, bv2, ST, G)
    return logits


# speedup vs baseline: 1.9838x; 1.1890x over previous
import functools

import jax
import jax.numpy as jnp
from jax import lax
from jax.experimental import pallas as pl
from jax.experimental.pallas import tpu as pltpu
from jax.experimental.pallas import tpu_sc as plsc

N = 10000
E = 320000
HD = 128
H = 8
OUT = 16
NEG = 0.2

RB = 1000
NRB = N // RB

NC = 2
NS = 16
NW = NC * NS
EPW = E // NW
CH = 40
NCHUNK = EPW // CH
DEPTH = 4
BLK = 20
NBLK = 12
NPAD = 10240
RPT = NPAD // NS
CPR = 128
AW = HD + 16


def _proj_and_shift(feat, alv, arv, S16, elp_ref, rp_ref, c_ref, mx_ref, i):
    el = jnp.dot(feat * alv, S16, preferred_element_type=jnp.float32)
    er = jnp.dot(feat * arv, S16, preferred_element_type=jnp.float32)
    elp_ref[...] = el
    rp_ref[...] = er
    bmax0 = jnp.max(el, axis=0, keepdims=True)
    bmax1 = jnp.max(er, axis=0, keepdims=True)

    @pl.when(i == 0)
    def _():
        mx_ref[0:1] = bmax0
        mx_ref[1:2] = bmax1

    @pl.when(i > 0)
    def _():
        mx_ref[0:1] = jnp.maximum(mx_ref[0:1], bmax0)
        mx_ref[1:2] = jnp.maximum(mx_ref[1:2], bmax1)

    @pl.when(i == NRB - 1)
    def _():
        s = mx_ref[0:1] + mx_ref[1:2]
        c_ref[...] = jnp.where(s > 0, s, s * NEG)


def _pre_body(x_ref, W_ref, alv_ref, arv_ref, S16_ref,
              feat_ref, elp_ref, rp_ref, c_ref, mx_ref):
    i = pl.program_id(0)
    feat = jnp.dot(x_ref[...], W_ref[...], preferred_element_type=jnp.float32)
    feat_ref[...] = feat
    _proj_and_shift(feat, alv_ref[...], arv_ref[...], S16_ref[...],
                    elp_ref, rp_ref, c_ref, mx_ref, i)


def _combine(p0, p1, hprev, bvec, ST):
    r = p0 + p1
    acc = r[:, 0:HD]
    den = r[:, HD:AW]
    inv = 1.0 / (den + 1e-30)
    inv128 = jnp.dot(inv, ST, preferred_element_type=jnp.float32)
    return acc * inv128 + hprev + bvec


def _mid_body(p0_ref, p1_ref, hprev_ref, bvec_ref, W_ref, alv_ref, arv_ref,
              S16_ref, ST_ref,
              feat_ref, elp_ref, rp_ref, c_ref, hn_ref, mx_ref):
    i = pl.program_id(0)
    pre = _combine(p0_ref[...], p1_ref[...], hprev_ref[...], bvec_ref[...],
                   ST_ref[...])
    hn = jnp.where(pre > 0, pre, jnp.exp(jnp.minimum(pre, 0.0)) - 1.0)
    hn_ref[...] = hn
    feat = jnp.dot(hn, W_ref[...], preferred_element_type=jnp.float32)
    feat_ref[...] = feat
    _proj_and_shift(feat, alv_ref[...], arv_ref[...], S16_ref[...],
                    elp_ref, rp_ref, c_ref, mx_ref, i)


def _post_body(p0_ref, p1_ref, h2_ref, bvec_ref, ST_ref, G_ref, out_ref):
    final = _combine(p0_ref[...], p1_ref[...], h2_ref[...], bvec_ref[...],
                     ST_ref[...])
    out_ref[...] = jnp.dot(final, G_ref[...], preferred_element_type=jnp.float32)


def _row_spec(cols):
    return pl.BlockSpec((RB, cols), lambda i: (i, 0))


def _full_spec(rows, cols):
    return pl.BlockSpec((rows, cols), lambda i: (0, 0))


_F32 = jnp.float32


def _tc_pre(x, W, alv, arv, S16):
    return pl.pallas_call(
        _pre_body,
        grid=(NRB,),
        in_specs=[_row_spec(HD), _full_spec(HD, HD), _full_spec(1, HD),
                  _full_spec(1, HD), _full_spec(HD, 16)],
        out_specs=[_row_spec(HD), _row_spec(16), _row_spec(16),
                   _full_spec(1, 16)],
        out_shape=[jax.ShapeDtypeStruct((N, HD), _F32),
                   jax.ShapeDtypeStruct((N, 16), _F32),
                   jax.ShapeDtypeStruct((N, 16), _F32),
                   jax.ShapeDtypeStruct((1, 16), _F32)],
        scratch_shapes=[pltpu.VMEM((2, 16), _F32)],
    )(x, W, alv, arv, S16)


def _tc_mid(p0, p1, hprev, bvec, W, alv, arv, S16, ST):
    return pl.pallas_call(
        _mid_body,
        grid=(NRB,),
        in_specs=[_row_spec(AW), _row_spec(AW), _row_spec(HD),
                  _full_spec(1, HD), _full_spec(HD, HD), _full_spec(1, HD),
                  _full_spec(1, HD), _full_spec(HD, 16), _full_spec(16, HD)],
        out_specs=[_row_spec(HD), _row_spec(16), _row_spec(16),
                   _full_spec(1, 16), _row_spec(HD)],
        out_shape=[jax.ShapeDtypeStruct((N, HD), _F32),
                   jax.ShapeDtypeStruct((N, 16), _F32),
                   jax.ShapeDtypeStruct((N, 16), _F32),
                   jax.ShapeDtypeStruct((1, 16), _F32),
                   jax.ShapeDtypeStruct((N, HD), _F32)],
        scratch_shapes=[pltpu.VMEM((2, 16), _F32)],
    )(p0, p1, hprev, bvec, W, alv, arv, S16, ST)


def _tc_post(p0, p1, h2, bvec, ST, G):
    return pl.pallas_call(
        _post_body,
        grid=(NRB,),
        in_specs=[_row_spec(AW), _row_spec(AW), _row_spec(HD),
                  _full_spec(1, HD), _full_spec(16, HD), _full_spec(HD, 16)],
        out_specs=_row_spec(16),
        out_shape=jax.ShapeDtypeStruct((N, 16), _F32),
    )(p0, p1, h2, bvec, ST, G)


def _sc_body(F_hbm, R_hbm, C_hbm, src_hbm, dst_hbm, P_hbm,
             idx_s, idx_d, sidx, Fc, Rc, M, cvb, A,
             semf0, semf1, semf2, semf3,
             semr0, semr1, semr2, semr3,
             semi0, semi1, semi2, semi3,
             semm0, semm1):
    cid = lax.axis_index("c")
    sid = lax.axis_index("s")
    wid = sid * NC + cid
    semf = (semf0, semf1, semf2, semf3)
    semr = (semr0, semr1, semr2, semr3)
    semi = (semi0, semi1, semi2, semi3)
    semm = (semm0, semm1)

    base0 = wid * EPW

    def _fetch_idx_sync(c, b):
        base = base0 + c * CH
        pltpu.sync_copy(src_hbm.at[pl.ds(base, CH)], idx_s.at[b])
        pltpu.sync_copy(dst_hbm.at[pl.ds(base, CH)], idx_d.at[b])

    def _fetch_idx_async(c, b):
        base = base0 + c * CH
        pltpu.async_copy(src_hbm.at[pl.ds(base, CH)], idx_s.at[b], semi[b])
        pltpu.async_copy(dst_hbm.at[pl.ds(base, CH)], idx_d.at[b], semi[b])

    def _wait_idx(b):
        pltpu.make_async_copy(src_hbm.at[pl.ds(0, CH)], idx_s.at[b],
                              semi[b]).wait()
        pltpu.make_async_copy(dst_hbm.at[pl.ds(0, CH)], idx_d.at[b],
                              semi[b]).wait()

    def _gather_rows(b):
        pltpu.async_copy(F_hbm.at[idx_s.at[b]], Fc.at[b], semf[b])
        pltpu.async_copy(R_hbm.at[idx_d.at[b]], Rc.at[b], semr[b])

    def _wait_scatter(m):
        pltpu.make_async_copy(M.at[m], A.at[sidx.at[m]], semm[m]).wait()

    for b in range(DEPTH - 1):
        _fetch_idx_sync(b, b)
        _gather_rows(b)
    _fetch_idx_async(DEPTH - 1, DEPTH - 1)

    def _zrow(i, _):
        for j in range(AW // 16):
            M[0, i, pl.ds(j * 16, 16)] = jnp.zeros((16,), _F32)
        return 0

    lax.fori_loop(0, CH, _zrow, 0)
    for j in range(RPT // CH):
        pltpu.sync_copy(M.at[0], A.at[pl.ds(sid * RPT + j * CH, CH)])
    pltpu.sync_copy(C_hbm, cvb)
    cv = cvb[...]
    plsc.subcore_barrier()

    def _chunk_body(c, b, m, may_gather, may_fetch, may_wait_scatter):
        pltpu.make_async_copy(F_hbm.at[idx_s.at[b]], Fc.at[b],
                              semf[b]).wait()
        pltpu.make_async_copy(R_hbm.at[idx_d.at[b]], Rc.at[b],
                              semr[b]).wait()
        if may_wait_scatter is True:
            _wait_scatter(m)
        elif may_wait_scatter is not False:
            pl.when(may_wait_scatter)(lambda: _wait_scatter(m))
        sidx[m, pl.ds(0, 16)] = idx_d[b, pl.ds(0, 16)]
        sidx[m, pl.ds(16, 16)] = idx_d[b, pl.ds(16, 16)]
        sidx[m, pl.ds(24, 16)] = idx_d[b, pl.ds(24, 16)]
        bp = (b - 1) % DEPTH

        def _do_gather():
            _wait_idx(bp)
            _gather_rows(bp)

        if may_gather is True:
            _do_gather()
        elif may_gather is not False:
            pl.when(may_gather)(_do_gather)
        if may_fetch is True:
            _fetch_idx_async(c + DEPTH, b)
        elif may_fetch is not False:
            pl.when(may_fetch)(lambda: _fetch_idx_async(c + DEPTH, b))

        def _edge(i, _):
            els = Fc[b, i, pl.ds(HD, 16)]
            erd = Rc[b, i, :]
            sv = els + erd
            e = jnp.where(sv > 0, sv, sv * NEG)
            ee = jnp.exp(e - cv)
            M[m, i, pl.ds(HD, 16)] = ee
            for h in range(H):
                M[m, i, pl.ds(h * OUT, OUT)] = (
                    Fc[b, i, pl.ds(h * OUT, OUT)] * ee[h])
            return 0

        lax.fori_loop(0, CH, _edge, 0)
        pltpu.async_copy(M.at[m], A.at[sidx.at[m]], semm[m], add=True)

    def _phase(g, _):
        for j in range(BLK):
            c = BLK * g + j
            _chunk_body(c, j % DEPTH, j % 2,
                        may_gather=c + DEPTH - 1 < NCHUNK,
                        may_fetch=c + DEPTH < NCHUNK,
                        may_wait_scatter=(True if j >= 2 else c >= 2))
        return 0

    lax.fori_loop(0, NBLK, _phase, 0)
    for j in range(NBLK * BLK, NCHUNK):
        _chunk_body(j, j % DEPTH, j % 2,
                    may_gather=j + DEPTH - 1 < NCHUNK,
                    may_fetch=j + DEPTH < NCHUNK,
                    may_wait_scatter=True)
    _wait_scatter(0)
    _wait_scatter(1)
    plsc.subcore_barrier()

    for j in range(RPT // CH):
        b = j % DEPTH
        r0 = sid * RPT + j * CH

        @pl.when(j >= DEPTH)
        def _():
            pltpu.make_async_copy(Fc.at[b], P_hbm.at[cid, pl.ds(0, CH)],
                                  semr[b]).wait()
        pltpu.make_async_copy(A.at[pl.ds(r0, CH)], Fc.at[b], semf[b]).start()
        pltpu.make_async_copy(A.at[pl.ds(r0, CH)], Fc.at[b], semf[b]).wait()
        pltpu.async_copy(Fc.at[b], P_hbm.at[cid, pl.ds(r0, CH)], semr[b])
    for j in range(RPT // CH - DEPTH, RPT // CH):
        b = j % DEPTH
        pltpu.make_async_copy(Fc.at[b], P_hbm.at[cid, pl.ds(0, CH)],
                              semr[b]).wait()


def _sc_edges(F, R, C, src, dst):
    mesh = plsc.VectorSubcoreMesh(core_axis_name="c", subcore_axis_name="s")
    k = pl.kernel(
        _sc_body,
        out_type=jax.ShapeDtypeStruct((NC, NPAD, AW), _F32),
        mesh=mesh,
        compiler_params=pltpu.CompilerParams(use_tc_tiling_on_sc=False),
        scratch_types=[
            pltpu.VMEM((DEPTH, CH), jnp.int32),
            pltpu.VMEM((DEPTH, CH), jnp.int32),
            pltpu.VMEM((2, CH), jnp.int32),
            pltpu.VMEM((DEPTH, CH, AW), _F32),
            pltpu.VMEM((DEPTH, CH, 16), _F32),
            pltpu.VMEM((2, CH, AW), _F32),
            pltpu.VMEM((16,), _F32),
            pltpu.VMEM_SHARED((NPAD, AW), _F32),
        ] + [pltpu.SemaphoreType.DMA] * 14,
    )
    return k(F, R, C, src, dst)


def kernel(inputs, edge_index, W0, al0, ar0, b0, W1, al1, ar1, b1,
           W2, al2, ar2, b2):
    x = inputs.astype(_F32)
    src = edge_index[0].astype(jnp.int32)
    dst = edge_index[1].astype(jnp.int32)

    j = jnp.arange(HD)
    S16 = (jnp.arange(16)[None, :] == (j // OUT)[:, None]).astype(_F32)
    ST = S16.T.copy()
    G = ((jnp.arange(16)[None, :] == (j % OUT)[:, None]).astype(_F32) / H)

    def layer_consts(al, ar, b):
        return (al.reshape(1, HD), ar.reshape(1, HD), b.reshape(1, HD))

    alv0, arv0, bv0 = layer_consts(al0, ar0, b0)
    alv1, arv1, bv1 = layer_consts(al1, ar1, b1)
    alv2, arv2, bv2 = layer_consts(al2, ar2, b2)

    feat, elp, rp, C = _tc_pre(x, W0, alv0, arv0, S16)
    F = jnp.concatenate([feat, elp], axis=1)
    P = _sc_edges(F, rp, C.reshape(16), src, dst)

    zeros_h = jnp.zeros((N, HD), _F32)
    feat, elp, rp, C, h1 = _tc_mid(P[0], P[1], zeros_h, bv0, W1, alv1, arv1,
                                   S16, ST)
    F = jnp.concatenate([feat, elp], axis=1)
    P = _sc_edges(F, rp, C.reshape(16), src, dst)

    feat, elp, rp, C, h2 = _tc_mid(P[0], P[1], h1, bv1, W2, alv2, arv2,
                                   S16, ST)
    F = jnp.concatenate([feat, elp], axis=1)
    P = _sc_edges(F, rp, C.reshape(16), src, dst)

    logits = _tc_post(P[0], P[1], h2, bv2, ST, G)
    return logits


# speedup vs baseline: 2.1306x; 1.0740x over previous
import functools

import jax
import jax.numpy as jnp
from jax import lax
from jax.experimental import pallas as pl
from jax.experimental.pallas import tpu as pltpu
from jax.experimental.pallas import tpu_sc as plsc

N = 10000
E = 320000
HD = 128
H = 8
OUT = 16
NEG = 0.2

RB = 1000
NRB = N // RB

NC = 2
NS = 16
NW = NC * NS
EPW = E // NW
CH = 40
NCHUNK = EPW // CH
DEPTH = 4
BLK = 20
NBLK = 12
NPAD = 10240
RPT = NPAD // NS
CPR = 128
AW = HD + 16


def _proj_and_shift(feat, alv, arv, S16, f_ref, rp_ref, c_ref, mx_ref, i):
    el = jnp.dot(feat * alv, S16, preferred_element_type=jnp.float32)
    er = jnp.dot(feat * arv, S16, preferred_element_type=jnp.float32)
    f_ref[:, 0:HD] = feat
    f_ref[:, HD:AW] = el
    rp_ref[...] = er
    bmax0 = jnp.max(el, axis=0, keepdims=True)
    bmax1 = jnp.max(er, axis=0, keepdims=True)

    @pl.when(i == 0)
    def _():
        mx_ref[0:1] = bmax0
        mx_ref[1:2] = bmax1

    @pl.when(i > 0)
    def _():
        mx_ref[0:1] = jnp.maximum(mx_ref[0:1], bmax0)
        mx_ref[1:2] = jnp.maximum(mx_ref[1:2], bmax1)

    @pl.when(i == NRB - 1)
    def _():
        s = mx_ref[0:1] + mx_ref[1:2]
        c_ref[...] = jnp.where(s > 0, s, s * NEG)


def _pre_body(x_ref, W_ref, alv_ref, arv_ref, S16_ref,
              f_ref, rp_ref, c_ref, mx_ref):
    i = pl.program_id(0)
    feat = jnp.dot(x_ref[...], W_ref[...], preferred_element_type=jnp.float32)
    _proj_and_shift(feat, alv_ref[...], arv_ref[...], S16_ref[...],
                    f_ref, rp_ref, c_ref, mx_ref, i)


def _combine(p0, p1, hprev, bvec, ST):
    r = p0[0] + p1[0]
    acc = r[:, 0:HD]
    den = r[:, HD:AW]
    inv = 1.0 / (den + 1e-30)
    inv128 = jnp.dot(inv, ST, preferred_element_type=jnp.float32)
    return acc * inv128 + hprev + bvec


def _mid_body(p0_ref, p1_ref, hprev_ref, bvec_ref, W_ref, alv_ref, arv_ref,
              S16_ref, ST_ref,
              f_ref, rp_ref, c_ref, hn_ref, mx_ref):
    i = pl.program_id(0)
    pre = _combine(p0_ref[...], p1_ref[...], hprev_ref[...], bvec_ref[...],
                   ST_ref[...])
    hn = jnp.where(pre > 0, pre, jnp.exp(jnp.minimum(pre, 0.0)) - 1.0)
    hn_ref[...] = hn
    feat = jnp.dot(hn, W_ref[...], preferred_element_type=jnp.float32)
    _proj_and_shift(feat, alv_ref[...], arv_ref[...], S16_ref[...],
                    f_ref, rp_ref, c_ref, mx_ref, i)


def _post_body(p0_ref, p1_ref, h2_ref, bvec_ref, ST_ref, G_ref, out_ref):
    final = _combine(p0_ref[...], p1_ref[...], h2_ref[...], bvec_ref[...],
                     ST_ref[...])
    out_ref[...] = jnp.dot(final, G_ref[...], preferred_element_type=jnp.float32)


def _row_spec(cols):
    return pl.BlockSpec((RB, cols), lambda i: (i, 0))


def _p_spec(which):
    return pl.BlockSpec((1, RB, AW), lambda i, w=which: (w, i, 0))


def _full_spec(rows, cols):
    return pl.BlockSpec((rows, cols), lambda i: (0, 0))


_F32 = jnp.float32


def _tc_pre(x, W, alv, arv, S16):
    return pl.pallas_call(
        _pre_body,
        grid=(NRB,),
        in_specs=[_row_spec(HD), _full_spec(HD, HD), _full_spec(1, HD),
                  _full_spec(1, HD), _full_spec(HD, 16)],
        out_specs=[_row_spec(AW), _row_spec(16), _full_spec(1, 16)],
        out_shape=[jax.ShapeDtypeStruct((N, AW), _F32),
                   jax.ShapeDtypeStruct((N, 16), _F32),
                   jax.ShapeDtypeStruct((1, 16), _F32)],
        scratch_shapes=[pltpu.VMEM((2, 16), _F32)],
    )(x, W, alv, arv, S16)


def _tc_mid(P, hprev, bvec, W, alv, arv, S16, ST):
    return pl.pallas_call(
        _mid_body,
        grid=(NRB,),
        in_specs=[_p_spec(0), _p_spec(1), _row_spec(HD),
                  _full_spec(1, HD), _full_spec(HD, HD), _full_spec(1, HD),
                  _full_spec(1, HD), _full_spec(HD, 16), _full_spec(16, HD)],
        out_specs=[_row_spec(AW), _row_spec(16), _full_spec(1, 16),
                   _row_spec(HD)],
        out_shape=[jax.ShapeDtypeStruct((N, AW), _F32),
                   jax.ShapeDtypeStruct((N, 16), _F32),
                   jax.ShapeDtypeStruct((1, 16), _F32),
                   jax.ShapeDtypeStruct((N, HD), _F32)],
        scratch_shapes=[pltpu.VMEM((2, 16), _F32)],
    )(P, P, hprev, bvec, W, alv, arv, S16, ST)


def _tc_post(P, h2, bvec, ST, G):
    return pl.pallas_call(
        _post_body,
        grid=(NRB,),
        in_specs=[_p_spec(0), _p_spec(1), _row_spec(HD),
                  _full_spec(1, HD), _full_spec(16, HD), _full_spec(HD, 16)],
        out_specs=_row_spec(16),
        out_shape=jax.ShapeDtypeStruct((N, 16), _F32),
    )(P, P, h2, bvec, ST, G)


def _sc_body(F_hbm, R_hbm, C_hbm, src_hbm, dst_hbm, P_hbm,
             idx_s, idx_d, sidx, Fc, Rc, M, cvb, A,
             semf0, semf1, semf2, semf3,
             semr0, semr1, semr2, semr3,
             semi0, semi1, semi2, semi3,
             semm0, semm1):
    cid = lax.axis_index("c")
    sid = lax.axis_index("s")
    wid = sid * NC + cid
    semf = (semf0, semf1, semf2, semf3)
    semr = (semr0, semr1, semr2, semr3)
    semi = (semi0, semi1, semi2, semi3)
    semm = (semm0, semm1)

    base0 = wid * EPW

    def _fetch_idx_sync(c, b):
        base = base0 + c * CH
        pltpu.sync_copy(src_hbm.at[pl.ds(base, CH)], idx_s.at[b])
        pltpu.sync_copy(dst_hbm.at[pl.ds(base, CH)], idx_d.at[b])

    def _fetch_idx_async(c, b):
        base = base0 + c * CH
        pltpu.async_copy(src_hbm.at[pl.ds(base, CH)], idx_s.at[b], semi[b])
        pltpu.async_copy(dst_hbm.at[pl.ds(base, CH)], idx_d.at[b], semi[b])

    def _wait_idx(b):
        pltpu.make_async_copy(src_hbm.at[pl.ds(0, CH)], idx_s.at[b],
                              semi[b]).wait()
        pltpu.make_async_copy(dst_hbm.at[pl.ds(0, CH)], idx_d.at[b],
                              semi[b]).wait()

    def _gather_rows(b):
        pltpu.async_copy(F_hbm.at[idx_s.at[b]], Fc.at[b], semf[b])
        pltpu.async_copy(R_hbm.at[idx_d.at[b]], Rc.at[b], semr[b])

    def _wait_scatter(m):
        pltpu.make_async_copy(M.at[m], A.at[sidx.at[m]], semm[m]).wait()

    for b in range(DEPTH - 1):
        _fetch_idx_sync(b, b)
        _gather_rows(b)
    _fetch_idx_async(DEPTH - 1, DEPTH - 1)

    def _zrow(i, _):
        for j in range(AW // 16):
            M[0, i, pl.ds(j * 16, 16)] = jnp.zeros((16,), _F32)
        return 0

    lax.fori_loop(0, CH, _zrow, 0)
    for j in range(RPT // CH):
        pltpu.sync_copy(M.at[0], A.at[pl.ds(sid * RPT + j * CH, CH)])
    pltpu.sync_copy(C_hbm, cvb)
    cv = cvb[...]
    plsc.subcore_barrier()

    def _chunk_body(c, b, m, may_gather, may_fetch, may_wait_scatter):
        pltpu.make_async_copy(F_hbm.at[idx_s.at[b]], Fc.at[b],
                              semf[b]).wait()
        pltpu.make_async_copy(R_hbm.at[idx_d.at[b]], Rc.at[b],
                              semr[b]).wait()
        if may_wait_scatter is True:
            _wait_scatter(m)
        elif may_wait_scatter is not False:
            pl.when(may_wait_scatter)(lambda: _wait_scatter(m))
        sidx[m, pl.ds(0, 16)] = idx_d[b, pl.ds(0, 16)]
        sidx[m, pl.ds(16, 16)] = idx_d[b, pl.ds(16, 16)]
        sidx[m, pl.ds(24, 16)] = idx_d[b, pl.ds(24, 16)]
        bp = (b - 1) % DEPTH

        def _do_gather():
            _wait_idx(bp)
            _gather_rows(bp)

        if may_gather is True:
            _do_gather()
        elif may_gather is not False:
            pl.when(may_gather)(_do_gather)
        if may_fetch is True:
            _fetch_idx_async(c + DEPTH, b)
        elif may_fetch is not False:
            pl.when(may_fetch)(lambda: _fetch_idx_async(c + DEPTH, b))

        def _edge(i, _):
            els = Fc[b, i, pl.ds(HD, 16)]
            erd = Rc[b, i, :]
            sv = els + erd
            e = jnp.where(sv > 0, sv, sv * NEG)
            ee = jnp.exp(e - cv)
            M[m, i, pl.ds(HD, 16)] = ee
            for h in range(H):
                M[m, i, pl.ds(h * OUT, OUT)] = (
                    Fc[b, i, pl.ds(h * OUT, OUT)] * ee[h])
            return 0

        lax.fori_loop(0, CH, _edge, 0)
        pltpu.async_copy(M.at[m], A.at[sidx.at[m]], semm[m], add=True)

    def _phase(g, _):
        for j in range(BLK):
            c = BLK * g + j
            _chunk_body(c, j % DEPTH, j % 2,
                        may_gather=c + DEPTH - 1 < NCHUNK,
                        may_fetch=c + DEPTH < NCHUNK,
                        may_wait_scatter=(True if j >= 2 else c >= 2))
        return 0

    lax.fori_loop(0, NBLK, _phase, 0)
    for j in range(NBLK * BLK, NCHUNK):
        _chunk_body(j, j % DEPTH, j % 2,
                    may_gather=j + DEPTH - 1 < NCHUNK,
                    may_fetch=j + DEPTH < NCHUNK,
                    may_wait_scatter=True)
    _wait_scatter(0)
    _wait_scatter(1)
    plsc.subcore_barrier()

    for j in range(RPT // CH):
        b = j % DEPTH
        r0 = sid * RPT + j * CH

        @pl.when(j >= DEPTH)
        def _():
            pltpu.make_async_copy(Fc.at[b], P_hbm.at[cid, pl.ds(0, CH)],
                                  semr[b]).wait()
        pltpu.make_async_copy(A.at[pl.ds(r0, CH)], Fc.at[b], semf[b]).start()
        pltpu.make_async_copy(A.at[pl.ds(r0, CH)], Fc.at[b], semf[b]).wait()
        pltpu.async_copy(Fc.at[b], P_hbm.at[cid, pl.ds(r0, CH)], semr[b])
    for j in range(RPT // CH - DEPTH, RPT // CH):
        b = j % DEPTH
        pltpu.make_async_copy(Fc.at[b], P_hbm.at[cid, pl.ds(0, CH)],
                              semr[b]).wait()


def _sc_edges(F, R, C, src, dst):
    mesh = plsc.VectorSubcoreMesh(core_axis_name="c", subcore_axis_name="s")
    k = pl.kernel(
        _sc_body,
        out_type=jax.ShapeDtypeStruct((NC, NPAD, AW), _F32),
        mesh=mesh,
        compiler_params=pltpu.CompilerParams(use_tc_tiling_on_sc=False),
        scratch_types=[
            pltpu.VMEM((DEPTH, CH), jnp.int32),
            pltpu.VMEM((DEPTH, CH), jnp.int32),
            pltpu.VMEM((2, CH), jnp.int32),
            pltpu.VMEM((DEPTH, CH, AW), _F32),
            pltpu.VMEM((DEPTH, CH, 16), _F32),
            pltpu.VMEM((2, CH, AW), _F32),
            pltpu.VMEM((16,), _F32),
            pltpu.VMEM_SHARED((NPAD, AW), _F32),
        ] + [pltpu.SemaphoreType.DMA] * 14,
    )
    return k(F, R, C, src, dst)


def kernel(inputs, edge_index, W0, al0, ar0, b0, W1, al1, ar1, b1,
           W2, al2, ar2, b2):
    x = inputs.astype(_F32)
    src = edge_index[0].astype(jnp.int32)
    dst = edge_index[1].astype(jnp.int32)

    j = jnp.arange(HD)
    S16 = (jnp.arange(16)[None, :] == (j // OUT)[:, None]).astype(_F32)
    ST = S16.T.copy()
    G = ((jnp.arange(16)[None, :] == (j % OUT)[:, None]).astype(_F32) / H)

    def layer_consts(al, ar, b):
        return (al.reshape(1, HD), ar.reshape(1, HD), b.reshape(1, HD))

    alv0, arv0, bv0 = layer_consts(al0, ar0, b0)
    alv1, arv1, bv1 = layer_consts(al1, ar1, b1)
    alv2, arv2, bv2 = layer_consts(al2, ar2, b2)

    F, rp, C = _tc_pre(x, W0, alv0, arv0, S16)
    P = _sc_edges(F, rp, C.reshape(16), src, dst)

    zeros_h = jnp.zeros((N, HD), _F32)
    F, rp, C, h1 = _tc_mid(P, zeros_h, bv0, W1, alv1, arv1, S16, ST)
    P = _sc_edges(F, rp, C.reshape(16), src, dst)

    F, rp, C, h2 = _tc_mid(P, h1, bv1, W2, alv2, arv2, S16, ST)
    P = _sc_edges(F, rp, C.reshape(16), src, dst)

    logits = _tc_post(P, h2, bv2, ST, G)
    return logits


# speedup vs baseline: 2.1402x; 1.0045x over previous
import functools

import jax
import jax.numpy as jnp
from jax import lax
from jax.experimental import pallas as pl
from jax.experimental.pallas import tpu as pltpu
from jax.experimental.pallas import tpu_sc as plsc

N = 10000
E = 320000
HD = 128
H = 8
OUT = 16
NEG = 0.2

RB = 1000
NRB = N // RB

NC = 2
NS = 16
NW = NC * NS
EPW = E // NW
CH = 40
NCHUNK = EPW // CH
DEPTH = 4
BLK = 20
NBLK = 12
NPAD = 10240
RPT = NPAD // NS
CPR = 128
AW = HD + 16


def _proj_and_shift(feat, alv, arv, S16, f_ref, rp_ref, c_ref, mx_ref, i):
    el = jnp.dot(feat * alv, S16, preferred_element_type=jnp.float32)
    er = jnp.dot(feat * arv, S16, preferred_element_type=jnp.float32)
    f_ref[:, 0:HD] = feat
    f_ref[:, HD:AW] = el
    rp_ref[...] = er
    bmax0 = jnp.max(el, axis=0, keepdims=True)
    bmax1 = jnp.max(er, axis=0, keepdims=True)

    @pl.when(i == 0)
    def _():
        mx_ref[0:1] = bmax0
        mx_ref[1:2] = bmax1

    @pl.when(i > 0)
    def _():
        mx_ref[0:1] = jnp.maximum(mx_ref[0:1], bmax0)
        mx_ref[1:2] = jnp.maximum(mx_ref[1:2], bmax1)

    @pl.when(i == NRB - 1)
    def _():
        s = mx_ref[0:1] + mx_ref[1:2]
        c_ref[...] = jnp.where(s > 0, s, s * NEG)


def _pre_body(x_ref, W_ref, alv_ref, arv_ref, S16_ref,
              f_ref, rp_ref, c_ref, mx_ref):
    i = pl.program_id(0)
    feat = jnp.dot(x_ref[...], W_ref[...], preferred_element_type=jnp.float32)
    _proj_and_shift(feat, alv_ref[...], arv_ref[...], S16_ref[...],
                    f_ref, rp_ref, c_ref, mx_ref, i)


def _combine(p0, p1, hprev, bvec, ST):
    r = p0[0] + p1[0]
    acc = r[:, 0:HD]
    den = r[:, HD:AW]
    inv = 1.0 / (den + 1e-30)
    inv128 = jnp.dot(inv, ST, preferred_element_type=jnp.float32)
    return acc * inv128 + hprev + bvec


def _mid_body(p0_ref, p1_ref, hprev_ref, bvec_ref, W_ref, alv_ref, arv_ref,
              S16_ref, ST_ref,
              f_ref, rp_ref, c_ref, hn_ref, mx_ref):
    i = pl.program_id(0)
    pre = _combine(p0_ref[...], p1_ref[...], hprev_ref[...], bvec_ref[...],
                   ST_ref[...])
    hn = jnp.where(pre > 0, pre, jnp.exp(jnp.minimum(pre, 0.0)) - 1.0)
    hn_ref[...] = hn
    feat = jnp.dot(hn, W_ref[...], preferred_element_type=jnp.float32)
    _proj_and_shift(feat, alv_ref[...], arv_ref[...], S16_ref[...],
                    f_ref, rp_ref, c_ref, mx_ref, i)


def _post_body(p0_ref, p1_ref, h2_ref, bvec_ref, ST_ref, G_ref, out_ref):
    final = _combine(p0_ref[...], p1_ref[...], h2_ref[...], bvec_ref[...],
                     ST_ref[...])
    out_ref[...] = jnp.dot(final, G_ref[...], preferred_element_type=jnp.float32)


def _row_spec(cols):
    return pl.BlockSpec((RB, cols), lambda i: (i, 0))


def _p_spec(which):
    return pl.BlockSpec((1, RB, AW), lambda i, w=which: (w, i, 0))


def _full_spec(rows, cols):
    return pl.BlockSpec((rows, cols), lambda i: (0, 0))


_F32 = jnp.float32


def _tc_pre(x, W, alv, arv, S16):
    return pl.pallas_call(
        _pre_body,
        grid=(NRB,),
        in_specs=[_row_spec(HD), _full_spec(HD, HD), _full_spec(1, HD),
                  _full_spec(1, HD), _full_spec(HD, 16)],
        out_specs=[_row_spec(AW), _row_spec(16), _full_spec(1, 16)],
        out_shape=[jax.ShapeDtypeStruct((N, AW), _F32),
                   jax.ShapeDtypeStruct((N, 16), _F32),
                   jax.ShapeDtypeStruct((1, 16), _F32)],
        scratch_shapes=[pltpu.VMEM((2, 16), _F32)],
    )(x, W, alv, arv, S16)


def _tc_mid(P, hprev, bvec, W, alv, arv, S16, ST):
    return pl.pallas_call(
        _mid_body,
        grid=(NRB,),
        in_specs=[_p_spec(0), _p_spec(1), _row_spec(HD),
                  _full_spec(1, HD), _full_spec(HD, HD), _full_spec(1, HD),
                  _full_spec(1, HD), _full_spec(HD, 16), _full_spec(16, HD)],
        out_specs=[_row_spec(AW), _row_spec(16), _full_spec(1, 16),
                   _row_spec(HD)],
        out_shape=[jax.ShapeDtypeStruct((N, AW), _F32),
                   jax.ShapeDtypeStruct((N, 16), _F32),
                   jax.ShapeDtypeStruct((1, 16), _F32),
                   jax.ShapeDtypeStruct((N, HD), _F32)],
        scratch_shapes=[pltpu.VMEM((2, 16), _F32)],
    )(P, P, hprev, bvec, W, alv, arv, S16, ST)


def _tc_post(P, h2, bvec, ST, G):
    return pl.pallas_call(
        _post_body,
        grid=(NRB,),
        in_specs=[_p_spec(0), _p_spec(1), _row_spec(HD),
                  _full_spec(1, HD), _full_spec(16, HD), _full_spec(HD, 16)],
        out_specs=_row_spec(16),
        out_shape=jax.ShapeDtypeStruct((N, 16), _F32),
    )(P, P, h2, bvec, ST, G)


def _sc_body(F_hbm, R_hbm, C_hbm, src_hbm, dst_hbm, P_hbm,
             idx_s, idx_d, sidx, Fc, Rc, M, cvb, A,
             semf0, semf1, semf2, semf3,
             semr0, semr1, semr2, semr3,
             semi0, semi1, semi2, semi3,
             semm0, semm1):
    cid = lax.axis_index("c")
    sid = lax.axis_index("s")
    wid = sid * NC + cid
    semf = (semf0, semf1, semf2, semf3)
    semr = (semr0, semr1, semr2, semr3)
    semi = (semi0, semi1, semi2, semi3)
    semm = (semm0, semm1)

    base0 = wid * EPW

    def _fetch_idx_sync(c, b):
        base = base0 + c * CH
        pltpu.sync_copy(src_hbm.at[pl.ds(base, CH)], idx_s.at[b])
        pltpu.sync_copy(dst_hbm.at[pl.ds(base, CH)], idx_d.at[b])

    def _fetch_idx_async(c, b):
        base = base0 + c * CH
        pltpu.async_copy(src_hbm.at[pl.ds(base, CH)], idx_s.at[b], semi[b])
        pltpu.async_copy(dst_hbm.at[pl.ds(base, CH)], idx_d.at[b], semi[b])

    def _wait_idx(b):
        pltpu.make_async_copy(src_hbm.at[pl.ds(0, CH)], idx_s.at[b],
                              semi[b]).wait()
        pltpu.make_async_copy(dst_hbm.at[pl.ds(0, CH)], idx_d.at[b],
                              semi[b]).wait()

    def _gather_rows(b):
        pltpu.async_copy(F_hbm.at[idx_s.at[b]], Fc.at[b], semf[b])
        pltpu.async_copy(R_hbm.at[idx_d.at[b]], Rc.at[b], semr[b])

    def _wait_scatter(m):
        pltpu.make_async_copy(M.at[m], A.at[sidx.at[m]], semm[m]).wait()

    for b in range(DEPTH - 1):
        _fetch_idx_sync(b, b)
        _gather_rows(b)
    _fetch_idx_async(DEPTH - 1, DEPTH - 1)

    def _zrow(i, _):
        for j in range(AW // 16):
            M[0, i, pl.ds(j * 16, 16)] = jnp.zeros((16,), _F32)
        return 0

    lax.fori_loop(0, CH, _zrow, 0)
    for j in range(RPT // CH):
        pltpu.async_copy(M.at[0], A.at[pl.ds(sid * RPT + j * CH, CH)],
                         semm[j % 2])
    pltpu.sync_copy(C_hbm, cvb)
    cv = cvb[...]
    for j in range(RPT // CH):
        pltpu.make_async_copy(M.at[0], A.at[pl.ds(sid * RPT, CH)],
                              semm[j % 2]).wait()
    plsc.subcore_barrier()

    def _chunk_body(c, b, m, may_gather, may_fetch, may_wait_scatter):
        pltpu.make_async_copy(F_hbm.at[idx_s.at[b]], Fc.at[b],
                              semf[b]).wait()
        pltpu.make_async_copy(R_hbm.at[idx_d.at[b]], Rc.at[b],
                              semr[b]).wait()
        if may_wait_scatter is True:
            _wait_scatter(m)
        elif may_wait_scatter is not False:
            pl.when(may_wait_scatter)(lambda: _wait_scatter(m))
        sidx[m, pl.ds(0, 16)] = idx_d[b, pl.ds(0, 16)]
        sidx[m, pl.ds(16, 16)] = idx_d[b, pl.ds(16, 16)]
        sidx[m, pl.ds(24, 16)] = idx_d[b, pl.ds(24, 16)]
        bp = (b - 1) % DEPTH

        def _do_gather():
            _wait_idx(bp)
            _gather_rows(bp)

        if may_gather is True:
            _do_gather()
        elif may_gather is not False:
            pl.when(may_gather)(_do_gather)
        if may_fetch is True:
            _fetch_idx_async(c + DEPTH, b)
        elif may_fetch is not False:
            pl.when(may_fetch)(lambda: _fetch_idx_async(c + DEPTH, b))

        def _edge(i, _):
            els = Fc[b, i, pl.ds(HD, 16)]
            erd = Rc[b, i, :]
            sv = els + erd
            e = jnp.where(sv > 0, sv, sv * NEG)
            ee = jnp.exp(e - cv)
            M[m, i, pl.ds(HD, 16)] = ee
            for h in range(H):
                M[m, i, pl.ds(h * OUT, OUT)] = (
                    Fc[b, i, pl.ds(h * OUT, OUT)] * ee[h])
            return 0

        lax.fori_loop(0, CH, _edge, 0)
        pltpu.async_copy(M.at[m], A.at[sidx.at[m]], semm[m], add=True)

    def _phase(g, _):
        for j in range(BLK):
            c = BLK * g + j
            _chunk_body(c, j % DEPTH, j % 2,
                        may_gather=c + DEPTH - 1 < NCHUNK,
                        may_fetch=c + DEPTH < NCHUNK,
                        may_wait_scatter=(True if j >= 2 else c >= 2))
        return 0

    lax.fori_loop(0, NBLK, _phase, 0)
    for j in range(NBLK * BLK, NCHUNK):
        _chunk_body(j, j % DEPTH, j % 2,
                    may_gather=j + DEPTH - 1 < NCHUNK,
                    may_fetch=j + DEPTH < NCHUNK,
                    may_wait_scatter=True)
    _wait_scatter(0)
    _wait_scatter(1)
    plsc.subcore_barrier()

    for j in range(RPT // CH):
        b = j % DEPTH
        r0 = sid * RPT + j * CH

        @pl.when(j >= DEPTH)
        def _():
            pltpu.make_async_copy(Fc.at[b], P_hbm.at[cid, pl.ds(0, CH)],
                                  semr[b]).wait()
        pltpu.make_async_copy(A.at[pl.ds(r0, CH)], Fc.at[b], semf[b]).start()
        pltpu.make_async_copy(A.at[pl.ds(r0, CH)], Fc.at[b], semf[b]).wait()
        pltpu.async_copy(Fc.at[b], P_hbm.at[cid, pl.ds(r0, CH)], semr[b])
    for j in range(RPT // CH - DEPTH, RPT // CH):
        b = j % DEPTH
        pltpu.make_async_copy(Fc.at[b], P_hbm.at[cid, pl.ds(0, CH)],
                              semr[b]).wait()


def _sc_edges(F, R, C, src, dst):
    mesh = plsc.VectorSubcoreMesh(core_axis_name="c", subcore_axis_name="s")
    k = pl.kernel(
        _sc_body,
        out_type=jax.ShapeDtypeStruct((NC, NPAD, AW), _F32),
        mesh=mesh,
        compiler_params=pltpu.CompilerParams(use_tc_tiling_on_sc=False),
        scratch_types=[
            pltpu.VMEM((DEPTH, CH), jnp.int32),
            pltpu.VMEM((DEPTH, CH), jnp.int32),
            pltpu.VMEM((2, CH), jnp.int32),
            pltpu.VMEM((DEPTH, CH, AW), _F32),
            pltpu.VMEM((DEPTH, CH, 16), _F32),
            pltpu.VMEM((2, CH, AW), _F32),
            pltpu.VMEM((16,), _F32),
            pltpu.VMEM_SHARED((NPAD, AW), _F32),
        ] + [pltpu.SemaphoreType.DMA] * 14,
    )
    return k(F, R, C, src, dst)


def kernel(inputs, edge_index, W0, al0, ar0, b0, W1, al1, ar1, b1,
           W2, al2, ar2, b2):
    x = inputs.astype(_F32)
    src = edge_index[0].astype(jnp.int32)
    dst = edge_index[1].astype(jnp.int32)

    j = jnp.arange(HD)
    S16 = (jnp.arange(16)[None, :] == (j // OUT)[:, None]).astype(_F32)
    ST = S16.T.copy()
    G = ((jnp.arange(16)[None, :] == (j % OUT)[:, None]).astype(_F32) / H)

    def layer_consts(al, ar, b):
        return (al.reshape(1, HD), ar.reshape(1, HD), b.reshape(1, HD))

    alv0, arv0, bv0 = layer_consts(al0, ar0, b0)
    alv1, arv1, bv1 = layer_consts(al1, ar1, b1)
    alv2, arv2, bv2 = layer_consts(al2, ar2, b2)

    F, rp, C = _tc_pre(x, W0, alv0, arv0, S16)
    P = _sc_edges(F, rp, C.reshape(16), src, dst)

    zeros_h = jnp.zeros((N, HD), _F32)
    F, rp, C, h1 = _tc_mid(P, zeros_h, bv0, W1, alv1, arv1, S16, ST)
    P = _sc_edges(F, rp, C.reshape(16), src, dst)

    F, rp, C, h2 = _tc_mid(P, h1, bv1, W2, alv2, arv2, S16, ST)
    P = _sc_edges(F, rp, C.reshape(16), src, dst)

    logits = _tc_post(P, h2, bv2, ST, G)
    return logits
